# Initial kernel scaffold; baseline (speedup 1.0000x reference)
import jax
import jax.numpy as jnp
from jax.experimental import pallas as pl


def kernel(mu, lam, pi, pi_write, Wq, bq, Wk, bk, Wv, bv, Wo, bo, Ws1, bs1, Ws2, bs2, Wsr, bsr, scratch_init, gate):
    raise NotImplementedError("write your pallas kernel here")


# speedup vs baseline: 14.7934x; 14.7934x over previous
import functools
import math

import jax
import jax.numpy as jnp
from jax.experimental import pallas as pl

NEG = -1e30


def _dot(a, b):
    return jax.lax.dot_general(a, b, (((1,), (0,)), ((), ())),
                               preferred_element_type=jnp.float32)


def _level_sizes(T):
    sizes = [T]
    n = T
    while n > 1:
        n = (n + 1) // 2
        sizes.append(n)
    return sizes


def _prep_kernel(mu_r, lam_r, pi_r, wq_r, bq_r, wk_r, bk_r, wv_r, bv_r,
                 w1mu_r, w1lv_r, w1pi_r, bs1_r, ws2_r, bs2_r, wsr_r, bsr_r,
                 scr_r, scrT_r, q_o, k_o, v_o, s_o, sc_o, *, M):
    mu = mu_r[...]
    q = _dot(mu, wq_r[...]) + bq_r[...]
    k_o[...] = _dot(mu, wk_r[...]) + bk_r[...]
    v_o[...] = _dot(mu, wv_r[...]) + bv_r[...]
    q_o[...] = q

    lv = -jnp.log(jnp.maximum(lam_r[...], 1e-6))
    h = (_dot(mu, w1mu_r[...]) + _dot(lv, w1lv_r[...])
         + _dot(pi_r[...], w1pi_r[...]) + bs1_r[...])
    h = h * jax.nn.sigmoid(h)
    s = jax.nn.sigmoid(_dot(h, ws2_r[...]) + bs2_r[...])
    s_o[...] = s

    sq = _dot(q, wsr_r[...]) + bsr_r[...]
    sc_scores = _dot(sq, scrT_r[...]) * (1.0 / math.sqrt(M))
    m = jnp.max(sc_scores, axis=-1, keepdims=True)
    e = jnp.exp(sc_scores - m)
    attn = e / jnp.sum(e, axis=-1, keepdims=True)
    sc_o[...] = _dot(attn, scr_r[...])


def _tree_kernel(k_r, v_r, w_r, tkT_o, tv_o, *, T, M, NTOT):
    ck = k_r[0]
    cv = v_r[0]
    cw = jnp.broadcast_to(w_r[0], (T, M))
    tv_o[0, 0:T, :] = cv
    tkT_o[0, :, 0:T] = ck.T
    off = T
    n = T
    while n > 1:
        h = n // 2
        r_iota = jax.lax.broadcasted_iota(jnp.int32, (h, n), 0)
        c_iota = jax.lax.broadcasted_iota(jnp.int32, (h, n), 1)
        A = jnp.where((c_iota // 2) == r_iota, 1.0, 0.0)
        sk = _dot(A, cw * ck)
        sv = _dot(A, cw * cv)
        sw = _dot(A, cw)
        tot = sw + 1e-8
        inv = 1.0 / tot
        ck = sk * inv
        cv = sv * inv
        cw = tot
        tv_o[0, off:off + h, :] = cv
        tkT_o[0, :, off:off + h] = ck.T
        off += h
        n = h
    if off < NTOT:
        tv_o[0, off:NTOT, :] = jnp.zeros((NTOT - off, M), jnp.float32)
        tkT_o[0, :, off:NTOT] = jnp.zeros((M, NTOT - off), jnp.float32)


def _attn_kernel(q_r, s_r, sc_r, tkT_r, tv_r, wo_r, bo_r, o_r, *, TQ, M, Ns):
    i = pl.program_id(1)
    q = q_r[0]
    s = s_r[0]
    t_ids = jax.lax.broadcasted_iota(jnp.int32, (TQ, 1), 0) + i * TQ

    L = len(Ns)
    ss = s * float(L - 1)
    kaps = [jnp.exp(-0.5 * (float(l) - ss) ** 2) for l in range(L)]
    kden = kaps[0]
    for l in range(1, L):
        kden = kden + kaps[l]
    kinv = 1.0 / jnp.maximum(kden, 1e-8)

    inv_sqrt_m = 1.0 / math.sqrt(M)
    acc = jnp.zeros((TQ, M), jnp.float32)
    off = 0
    for l, N in enumerate(Ns):
        KT = tkT_r[0, :, off:off + N]
        V = tv_r[0, off:off + N, :]
        sc = _dot(q, KT) * inv_sqrt_m
        niota = jax.lax.broadcasted_iota(jnp.int32, (TQ, N), 1)
        sc = jnp.where(niota > t_ids, NEG, sc)
        if N > 4:
            work = sc
            m1 = jnp.max(work, axis=-1, keepdims=True)
            idx = jnp.min(jnp.where(work == m1, niota, N), axis=-1,
                          keepdims=True)
            oneh = niota == idx
            P = jnp.where(oneh, 1.0, 0.0)
            den = jnp.ones_like(m1)
            work = jnp.where(oneh, NEG, work)
            for _ in range(3):
                m = jnp.max(work, axis=-1, keepdims=True)
                idx = jnp.min(jnp.where(work == m, niota, N), axis=-1,
                              keepdims=True)
                oneh = niota == idx
                wgt = jnp.exp(m - m1)
                P = P + jnp.where(oneh, wgt, 0.0)
                den = den + wgt
                work = jnp.where(oneh, NEG, work)
            ctx = _dot(P, V)
        else:
            m = jnp.max(sc, axis=-1, keepdims=True)
            e = jnp.exp(sc - m)
            den = jnp.sum(e, axis=-1, keepdims=True)
            ctx = _dot(e, V)
        acc = acc + (kaps[l] * kinv / den) * ctx
        off += N

    comb = (1.0 - s) * acc + s * sc_r[0]
    o_r[0] = _dot(comb, wo_r[...]) + bo_r[...]


def kernel(mu, lam, pi, pi_write, Wq, bq, Wk, bk, Wv, bv, Wo, bo,
           Ws1, bs1, Ws2, bs2, Wsr, bsr, scratch_init, gate):
    B, T, D = mu.shape
    M = Wq.shape[1]
    H = Ws1.shape[1]
    P7 = pi.shape[-1]
    BT = B * T
    Ns = _level_sizes(T)
    NTOT_raw = sum(Ns)
    NTOT = ((NTOT_raw + 7) // 8) * 8

    TQ = min(256, T)
    f32 = jnp.float32

    mu2 = mu.reshape(BT, D)
    lam2 = lam.reshape(BT, D)
    pi8 = jnp.pad(pi.reshape(BT, P7), ((0, 0), (0, 8 - P7)))
    w1mu = Ws1[:D]
    w1lv = Ws1[D:2 * D]
    w1pi = jnp.pad(Ws1[2 * D:], ((0, 8 - P7), (0, 0)))
    scr0 = scratch_init[0]
    scrT = scr0.T
    g = jax.nn.sigmoid(gate)
    wog = Wo * g
    bog = (bo * g).reshape(1, D)

    def row_spec(w):
        return pl.BlockSpec((TQ, w), lambda i: (i, 0))

    def full_spec(shape):
        nd = len(shape)
        return pl.BlockSpec(shape, lambda i, _n=nd: (0,) * _n)

    grid_a = BT // TQ
    q2, k2, v2, s2, sc2 = pl.pallas_call(
        functools.partial(_prep_kernel, M=M),
        grid=(grid_a,),
        in_specs=[
            row_spec(D), row_spec(D), row_spec(8),
            full_spec((D, M)), full_spec((1, M)),
            full_spec((D, M)), full_spec((1, M)),
            full_spec((D, M)), full_spec((1, M)),
            full_spec((D, H)), full_spec((D, H)), full_spec((8, H)),
            full_spec((1, H)), full_spec((H, 1)), full_spec((1, 1)),
            full_spec((M, M)), full_spec((1, M)),
            full_spec(scr0.shape), full_spec(scrT.shape),
        ],
        out_specs=[row_spec(M), row_spec(M), row_spec(M), row_spec(1),
                   row_spec(M)],
        out_shape=[
            jax.ShapeDtypeStruct((BT, M), f32),
            jax.ShapeDtypeStruct((BT, M), f32),
            jax.ShapeDtypeStruct((BT, M), f32),
            jax.ShapeDtypeStruct((BT, 1), f32),
            jax.ShapeDtypeStruct((BT, M), f32),
        ],
    )(mu2, lam2, pi8,
      Wq, bq.reshape(1, M), Wk, bk.reshape(1, M), Wv, bv.reshape(1, M),
      w1mu, w1lv, w1pi, bs1.reshape(1, H), Ws2, bs2.reshape(1, 1),
      Wsr, bsr.reshape(1, M), scr0, scrT)

    k3 = k2.reshape(B, T, M)
    v3 = v2.reshape(B, T, M)
    w3 = pi_write

    tkT, tv = pl.pallas_call(
        functools.partial(_tree_kernel, T=T, M=M, NTOT=NTOT),
        grid=(B,),
        in_specs=[
            pl.BlockSpec((1, T, M), lambda b: (b, 0, 0)),
            pl.BlockSpec((1, T, M), lambda b: (b, 0, 0)),
            pl.BlockSpec((1, T, 1), lambda b: (b, 0, 0)),
        ],
        out_specs=[
            pl.BlockSpec((1, M, NTOT), lambda b: (b, 0, 0)),
            pl.BlockSpec((1, NTOT, M), lambda b: (b, 0, 0)),
        ],
        out_shape=[
            jax.ShapeDtypeStruct((B, M, NTOT), f32),
            jax.ShapeDtypeStruct((B, NTOT, M), f32),
        ],
    )(k3, v3, w3)

    q3 = q2.reshape(B, T, M)
    s3 = s2.reshape(B, T, 1)
    sc3 = sc2.reshape(B, T, M)

    out3 = pl.pallas_call(
        functools.partial(_attn_kernel, TQ=TQ, M=M, Ns=Ns),
        grid=(B, T // TQ),
        in_specs=[
            pl.BlockSpec((1, TQ, M), lambda b, i: (b, i, 0)),
            pl.BlockSpec((1, TQ, 1), lambda b, i: (b, i, 0)),
            pl.BlockSpec((1, TQ, M), lambda b, i: (b, i, 0)),
            pl.BlockSpec((1, M, NTOT), lambda b, i: (b, 0, 0)),
            pl.BlockSpec((1, NTOT, M), lambda b, i: (b, 0, 0)),
            pl.BlockSpec((M, D), lambda b, i: (0, 0)),
            pl.BlockSpec((1, D), lambda b, i: (0, 0)),
        ],
        out_specs=pl.BlockSpec((1, TQ, D), lambda b, i: (b, i, 0)),
        out_shape=jax.ShapeDtypeStruct((B, T, D), f32),
    )(q3, s3, sc3, tkT, tv, wog, bog)

    return out3, s2.reshape(B, T)


# speedup vs baseline: 19.2059x; 1.2983x over previous
import functools
import math

import jax
import jax.numpy as jnp
from jax.experimental import pallas as pl

NEG = -1e30


def _dot(a, b):
    return jax.lax.dot_general(a, b, (((1,), (0,)), ((), ())),
                               preferred_element_type=jnp.float32)


def _level_sizes(T):
    sizes = [T]
    n = T
    while n > 1:
        n = (n + 1) // 2
        sizes.append(n)
    return sizes


def _prep_kernel(mu_r, lam_r, pi_r, wq_r, bq_r, wk_r, bk_r, wv_r, bv_r,
                 w1mu_r, w1lv_r, w1pi_r, bs1_r, ws2_r, bs2_r, wsr_r, bsr_r,
                 scr_r, scrT_r, q_o, k_o, v_o, s_o, sc_o, *, M):
    mu = mu_r[...]
    q = _dot(mu, wq_r[...]) + bq_r[...]
    k_o[...] = _dot(mu, wk_r[...]) + bk_r[...]
    v_o[...] = _dot(mu, wv_r[...]) + bv_r[...]
    q_o[...] = q

    lv = -jnp.log(jnp.maximum(lam_r[...], 1e-6))
    h = (_dot(mu, w1mu_r[...]) + _dot(lv, w1lv_r[...])
         + _dot(pi_r[...], w1pi_r[...]) + bs1_r[...])
    h = h * jax.nn.sigmoid(h)
    s = jax.nn.sigmoid(_dot(h, ws2_r[...]) + bs2_r[...])
    s_o[...] = s

    sq = _dot(q, wsr_r[...]) + bsr_r[...]
    sc_scores = _dot(sq, scrT_r[...]) * (1.0 / math.sqrt(M))
    m = jnp.max(sc_scores, axis=-1, keepdims=True)
    e = jnp.exp(sc_scores - m)
    attn = e / jnp.sum(e, axis=-1, keepdims=True)
    sc_o[...] = _dot(attn, scr_r[...])


def _tree_kernel(k_r, v_r, w_r, tkT_o, tv_o, *, T, M, NTOT):
    ck = k_r[0]
    cv = v_r[0]
    cw = jnp.broadcast_to(w_r[0], (T, M))
    tv_o[0, 0:T, :] = cv
    tkT_o[0, :, 0:T] = ck.T
    off = T
    n = T
    while n > 1:
        h = n // 2
        r_iota = jax.lax.broadcasted_iota(jnp.int32, (h, n), 0)
        c_iota = jax.lax.broadcasted_iota(jnp.int32, (h, n), 1)
        A = jnp.where((c_iota // 2) == r_iota, 1.0, 0.0)
        sk = _dot(A, cw * ck)
        sv = _dot(A, cw * cv)
        sw = _dot(A, cw)
        tot = sw + 1e-8
        inv = 1.0 / tot
        ck = sk * inv
        cv = sv * inv
        cw = tot
        tv_o[0, off:off + h, :] = cv
        tkT_o[0, :, off:off + h] = ck.T
        off += h
        n = h
    if off < NTOT:
        tv_o[0, off:NTOT, :] = jnp.zeros((NTOT - off, M), jnp.float32)
        tkT_o[0, :, off:NTOT] = jnp.zeros((M, NTOT - off), jnp.float32)


def _attn_kernel(q_r, s_r, sc_r, tkT_r, tv_r, wo_r, bo_r, o_r, *, TQ, M, Ns):
    i = pl.program_id(1)
    q = q_r[0]
    s = s_r[0]
    t_ids = jax.lax.broadcasted_iota(jnp.int32, (TQ, 1), 0) + i * TQ

    L = len(Ns)
    ss = s * float(L - 1)
    kaps = [jnp.exp(-0.5 * (float(l) - ss) ** 2) for l in range(L)]
    kden = kaps[0]
    for l in range(1, L):
        kden = kden + kaps[l]
    kinv = 1.0 / jnp.maximum(kden, 1e-8)

    inv_sqrt_m = 1.0 / math.sqrt(M)
    acc = jnp.zeros((TQ, M), jnp.float32)
    off = 0
    for l, N in enumerate(Ns):
        KT = tkT_r[0, :, off:off + N]
        V = tv_r[0, off:off + N, :]
        sc = _dot(q, KT) * inv_sqrt_m
        niota = jax.lax.broadcasted_iota(jnp.int32, (TQ, N), 1)
        sc = jnp.where(niota > t_ids, NEG, sc)
        if N > 4:
            m1 = jnp.max(sc, axis=-1, keepdims=True)
            work = jnp.where(sc == m1, NEG, sc)
            m2 = jnp.max(work, axis=-1, keepdims=True)
            work = jnp.where(work == m2, NEG, work)
            m3 = jnp.max(work, axis=-1, keepdims=True)
            work = jnp.where(work == m3, NEG, work)
            m4 = jnp.max(work, axis=-1, keepdims=True)
            e = jnp.exp(sc - m1)
            P = jnp.where(sc >= m4, e, 0.0)
            den = jnp.sum(P, axis=-1, keepdims=True)
            ctx = _dot(P, V)
        else:
            m = jnp.max(sc, axis=-1, keepdims=True)
            e = jnp.exp(sc - m)
            den = jnp.sum(e, axis=-1, keepdims=True)
            ctx = _dot(e, V)
        acc = acc + (kaps[l] * kinv / den) * ctx
        off += N

    comb = (1.0 - s) * acc + s * sc_r[0]
    o_r[0] = _dot(comb, wo_r[...]) + bo_r[...]


def kernel(mu, lam, pi, pi_write, Wq, bq, Wk, bk, Wv, bv, Wo, bo,
           Ws1, bs1, Ws2, bs2, Wsr, bsr, scratch_init, gate):
    B, T, D = mu.shape
    M = Wq.shape[1]
    H = Ws1.shape[1]
    P7 = pi.shape[-1]
    BT = B * T
    Ns = _level_sizes(T)
    NTOT_raw = sum(Ns)
    NTOT = ((NTOT_raw + 7) // 8) * 8

    TQ = min(256, T)
    f32 = jnp.float32

    mu2 = mu.reshape(BT, D)
    lam2 = lam.reshape(BT, D)
    pi8 = jnp.pad(pi.reshape(BT, P7), ((0, 0), (0, 8 - P7)))
    w1mu = Ws1[:D]
    w1lv = Ws1[D:2 * D]
    w1pi = jnp.pad(Ws1[2 * D:], ((0, 8 - P7), (0, 0)))
    scr0 = scratch_init[0]
    scrT = scr0.T
    g = jax.nn.sigmoid(gate)
    wog = Wo * g
    bog = (bo * g).reshape(1, D)

    def row_spec(w):
        return pl.BlockSpec((TQ, w), lambda i: (i, 0))

    def full_spec(shape):
        nd = len(shape)
        return pl.BlockSpec(shape, lambda i, _n=nd: (0,) * _n)

    grid_a = BT // TQ
    q2, k2, v2, s2, sc2 = pl.pallas_call(
        functools.partial(_prep_kernel, M=M),
        grid=(grid_a,),
        in_specs=[
            row_spec(D), row_spec(D), row_spec(8),
            full_spec((D, M)), full_spec((1, M)),
            full_spec((D, M)), full_spec((1, M)),
            full_spec((D, M)), full_spec((1, M)),
            full_spec((D, H)), full_spec((D, H)), full_spec((8, H)),
            full_spec((1, H)), full_spec((H, 1)), full_spec((1, 1)),
            full_spec((M, M)), full_spec((1, M)),
            full_spec(scr0.shape), full_spec(scrT.shape),
        ],
        out_specs=[row_spec(M), row_spec(M), row_spec(M), row_spec(1),
                   row_spec(M)],
        out_shape=[
            jax.ShapeDtypeStruct((BT, M), f32),
            jax.ShapeDtypeStruct((BT, M), f32),
            jax.ShapeDtypeStruct((BT, M), f32),
            jax.ShapeDtypeStruct((BT, 1), f32),
            jax.ShapeDtypeStruct((BT, M), f32),
        ],
    )(mu2, lam2, pi8,
      Wq, bq.reshape(1, M), Wk, bk.reshape(1, M), Wv, bv.reshape(1, M),
      w1mu, w1lv, w1pi, bs1.reshape(1, H), Ws2, bs2.reshape(1, 1),
      Wsr, bsr.reshape(1, M), scr0, scrT)

    k3 = k2.reshape(B, T, M)
    v3 = v2.reshape(B, T, M)
    w3 = pi_write

    tkT, tv = pl.pallas_call(
        functools.partial(_tree_kernel, T=T, M=M, NTOT=NTOT),
        grid=(B,),
        in_specs=[
            pl.BlockSpec((1, T, M), lambda b: (b, 0, 0)),
            pl.BlockSpec((1, T, M), lambda b: (b, 0, 0)),
            pl.BlockSpec((1, T, 1), lambda b: (b, 0, 0)),
        ],
        out_specs=[
            pl.BlockSpec((1, M, NTOT), lambda b: (b, 0, 0)),
            pl.BlockSpec((1, NTOT, M), lambda b: (b, 0, 0)),
        ],
        out_shape=[
            jax.ShapeDtypeStruct((B, M, NTOT), f32),
            jax.ShapeDtypeStruct((B, NTOT, M), f32),
        ],
    )(k3, v3, w3)

    q3 = q2.reshape(B, T, M)
    s3 = s2.reshape(B, T, 1)
    sc3 = sc2.reshape(B, T, M)

    out3 = pl.pallas_call(
        functools.partial(_attn_kernel, TQ=TQ, M=M, Ns=Ns),
        grid=(B, T // TQ),
        in_specs=[
            pl.BlockSpec((1, TQ, M), lambda b, i: (b, i, 0)),
            pl.BlockSpec((1, TQ, 1), lambda b, i: (b, i, 0)),
            pl.BlockSpec((1, TQ, M), lambda b, i: (b, i, 0)),
            pl.BlockSpec((1, M, NTOT), lambda b, i: (b, 0, 0)),
            pl.BlockSpec((1, NTOT, M), lambda b, i: (b, 0, 0)),
            pl.BlockSpec((M, D), lambda b, i: (0, 0)),
            pl.BlockSpec((1, D), lambda b, i: (0, 0)),
        ],
        out_specs=pl.BlockSpec((1, TQ, D), lambda b, i: (b, i, 0)),
        out_shape=jax.ShapeDtypeStruct((B, T, D), f32),
    )(q3, s3, sc3, tkT, tv, wog, bog)

    return out3, s2.reshape(B, T)


# speedup vs baseline: 20.6553x; 1.0755x over previous
import functools
import math

import jax
import jax.numpy as jnp
from jax.experimental import pallas as pl

NEG = -1e30


def _dot(a, b):
    return jax.lax.dot_general(a, b, (((1,), (0,)), ((), ())),
                               preferred_element_type=jnp.float32)


def _level_sizes(T):
    sizes = [T]
    n = T
    while n > 1:
        n = (n + 1) // 2
        sizes.append(n)
    return sizes


def _prep_kernel(mu_r, lam_r, pi_r, wq_r, bq_r, wk_r, bk_r, wv_r, bv_r,
                 w1mu_r, w1lv_r, w1pi_r, bs1_r, ws2_r, bs2_r, wsr_r, bsr_r,
                 scr_r, scrT_r, q_o, k_o, v_o, s_o, sc_o, *, M):
    mu = mu_r[...]
    q = _dot(mu, wq_r[...]) + bq_r[...]
    k_o[...] = _dot(mu, wk_r[...]) + bk_r[...]
    v_o[...] = _dot(mu, wv_r[...]) + bv_r[...]
    q_o[...] = q

    lv = -jnp.log(jnp.maximum(lam_r[...], 1e-6))
    h = (_dot(mu, w1mu_r[...]) + _dot(lv, w1lv_r[...])
         + _dot(pi_r[...], w1pi_r[...]) + bs1_r[...])
    h = h * jax.nn.sigmoid(h)
    s = jax.nn.sigmoid(_dot(h, ws2_r[...]) + bs2_r[...])
    s_o[...] = s

    sq = _dot(q, wsr_r[...]) + bsr_r[...]
    sc_scores = _dot(sq, scrT_r[...]) * (1.0 / math.sqrt(M))
    m = jnp.max(sc_scores, axis=-1, keepdims=True)
    e = jnp.exp(sc_scores - m)
    attn = e / jnp.sum(e, axis=-1, keepdims=True)
    sc_o[...] = _dot(attn, scr_r[...])


def _tree_kernel(k_r, v_r, w_r, tkT_o, tv_o, *, T, M, NTOT):
    ck = k_r[0]
    cv = v_r[0]
    cw = jnp.broadcast_to(w_r[0], (T, M))
    tv_o[0, 0:T, :] = cv
    tkT_o[0, :, 0:T] = ck.T
    off = T
    n = T
    while n > 1:
        h = n // 2
        r_iota = jax.lax.broadcasted_iota(jnp.int32, (h, n), 0)
        c_iota = jax.lax.broadcasted_iota(jnp.int32, (h, n), 1)
        A = jnp.where((c_iota // 2) == r_iota, 1.0, 0.0)
        sk = _dot(A, cw * ck)
        sv = _dot(A, cw * cv)
        sw = _dot(A, cw)
        tot = sw + 1e-8
        inv = 1.0 / tot
        ck = sk * inv
        cv = sv * inv
        cw = tot
        tv_o[0, off:off + h, :] = cv
        tkT_o[0, :, off:off + h] = ck.T
        off += h
        n = h
    if off < NTOT:
        tv_o[0, off:NTOT, :] = jnp.zeros((NTOT - off, M), jnp.float32)
        tkT_o[0, :, off:NTOT] = jnp.zeros((M, NTOT - off), jnp.float32)


def _attn_kernel(q_r, s_r, sc_r, tkT_r, tv_r, wo_r, bo_r, o_r, *, TQ, M, Ns):
    i = pl.program_id(1)
    q = q_r[0]
    s = s_r[0]
    t_ids = jax.lax.broadcasted_iota(jnp.int32, (TQ, 1), 0) + i * TQ

    L = len(Ns)
    ss = s * float(L - 1)
    kaps = [jnp.exp(-0.5 * (float(l) - ss) ** 2) for l in range(L)]
    kden = kaps[0]
    for l in range(1, L):
        kden = kden + kaps[l]
    kinv = 1.0 / jnp.maximum(kden, 1e-8)

    q = q * (1.0 / math.sqrt(M))
    niota_full = jax.lax.broadcasted_iota(jnp.int32, (TQ, Ns[0]), 1)
    acc = jnp.zeros((TQ, M), jnp.float32)
    off = 0
    for l, N in enumerate(Ns):
        KT = tkT_r[0, :, off:off + N]
        V = tv_r[0, off:off + N, :]
        sc = jnp.where(niota_full[:, :N] > t_ids, NEG, _dot(q, KT))
        if N > 4:
            m1 = jnp.max(sc, axis=-1, keepdims=True)
            m2 = jnp.max(jnp.where(sc >= m1, NEG, sc), axis=-1, keepdims=True)
            m3 = jnp.max(jnp.where(sc >= m2, NEG, sc), axis=-1, keepdims=True)
            m4 = jnp.max(jnp.where(sc >= m3, NEG, sc), axis=-1, keepdims=True)
            e = jnp.exp(sc - m1)
            P = jnp.where(sc >= m4, e, 0.0)
            den = jnp.sum(P, axis=-1, keepdims=True)
            ctx = _dot(P, V)
        else:
            m = jnp.max(sc, axis=-1, keepdims=True)
            e = jnp.exp(sc - m)
            den = jnp.sum(e, axis=-1, keepdims=True)
            ctx = _dot(e, V)
        acc = acc + (kaps[l] * kinv / den) * ctx
        off += N

    comb = (1.0 - s) * acc + s * sc_r[0]
    o_r[0] = _dot(comb, wo_r[...]) + bo_r[...]


def kernel(mu, lam, pi, pi_write, Wq, bq, Wk, bk, Wv, bv, Wo, bo,
           Ws1, bs1, Ws2, bs2, Wsr, bsr, scratch_init, gate):
    B, T, D = mu.shape
    M = Wq.shape[1]
    H = Ws1.shape[1]
    P7 = pi.shape[-1]
    BT = B * T
    Ns = _level_sizes(T)
    NTOT_raw = sum(Ns)
    NTOT = ((NTOT_raw + 7) // 8) * 8

    TQ = min(256, T)
    f32 = jnp.float32

    mu2 = mu.reshape(BT, D)
    lam2 = lam.reshape(BT, D)
    pi8 = jnp.pad(pi.reshape(BT, P7), ((0, 0), (0, 8 - P7)))
    w1mu = Ws1[:D]
    w1lv = Ws1[D:2 * D]
    w1pi = jnp.pad(Ws1[2 * D:], ((0, 8 - P7), (0, 0)))
    scr0 = scratch_init[0]
    scrT = scr0.T
    g = jax.nn.sigmoid(gate)
    wog = Wo * g
    bog = (bo * g).reshape(1, D)

    def row_spec(w):
        return pl.BlockSpec((TQ, w), lambda i: (i, 0))

    def full_spec(shape):
        nd = len(shape)
        return pl.BlockSpec(shape, lambda i, _n=nd: (0,) * _n)

    grid_a = BT // TQ
    q2, k2, v2, s2, sc2 = pl.pallas_call(
        functools.partial(_prep_kernel, M=M),
        grid=(grid_a,),
        in_specs=[
            row_spec(D), row_spec(D), row_spec(8),
            full_spec((D, M)), full_spec((1, M)),
            full_spec((D, M)), full_spec((1, M)),
            full_spec((D, M)), full_spec((1, M)),
            full_spec((D, H)), full_spec((D, H)), full_spec((8, H)),
            full_spec((1, H)), full_spec((H, 1)), full_spec((1, 1)),
            full_spec((M, M)), full_spec((1, M)),
            full_spec(scr0.shape), full_spec(scrT.shape),
        ],
        out_specs=[row_spec(M), row_spec(M), row_spec(M), row_spec(1),
                   row_spec(M)],
        out_shape=[
            jax.ShapeDtypeStruct((BT, M), f32),
            jax.ShapeDtypeStruct((BT, M), f32),
            jax.ShapeDtypeStruct((BT, M), f32),
            jax.ShapeDtypeStruct((BT, 1), f32),
            jax.ShapeDtypeStruct((BT, M), f32),
        ],
    )(mu2, lam2, pi8,
      Wq, bq.reshape(1, M), Wk, bk.reshape(1, M), Wv, bv.reshape(1, M),
      w1mu, w1lv, w1pi, bs1.reshape(1, H), Ws2, bs2.reshape(1, 1),
      Wsr, bsr.reshape(1, M), scr0, scrT)

    k3 = k2.reshape(B, T, M)
    v3 = v2.reshape(B, T, M)
    w3 = pi_write

    tkT, tv = pl.pallas_call(
        functools.partial(_tree_kernel, T=T, M=M, NTOT=NTOT),
        grid=(B,),
        in_specs=[
            pl.BlockSpec((1, T, M), lambda b: (b, 0, 0)),
            pl.BlockSpec((1, T, M), lambda b: (b, 0, 0)),
            pl.BlockSpec((1, T, 1), lambda b: (b, 0, 0)),
        ],
        out_specs=[
            pl.BlockSpec((1, M, NTOT), lambda b: (b, 0, 0)),
            pl.BlockSpec((1, NTOT, M), lambda b: (b, 0, 0)),
        ],
        out_shape=[
            jax.ShapeDtypeStruct((B, M, NTOT), f32),
            jax.ShapeDtypeStruct((B, NTOT, M), f32),
        ],
    )(k3, v3, w3)

    q3 = q2.reshape(B, T, M)
    s3 = s2.reshape(B, T, 1)
    sc3 = sc2.reshape(B, T, M)

    out3 = pl.pallas_call(
        functools.partial(_attn_kernel, TQ=TQ, M=M, Ns=Ns),
        grid=(B, T // TQ),
        in_specs=[
            pl.BlockSpec((1, TQ, M), lambda b, i: (b, i, 0)),
            pl.BlockSpec((1, TQ, 1), lambda b, i: (b, i, 0)),
            pl.BlockSpec((1, TQ, M), lambda b, i: (b, i, 0)),
            pl.BlockSpec((1, M, NTOT), lambda b, i: (b, 0, 0)),
            pl.BlockSpec((1, NTOT, M), lambda b, i: (b, 0, 0)),
            pl.BlockSpec((M, D), lambda b, i: (0, 0)),
            pl.BlockSpec((1, D), lambda b, i: (0, 0)),
        ],
        out_specs=pl.BlockSpec((1, TQ, D), lambda b, i: (b, i, 0)),
        out_shape=jax.ShapeDtypeStruct((B, T, D), f32),
    )(q3, s3, sc3, tkT, tv, wog, bog)

    return out3, s2.reshape(B, T)


# speedup vs baseline: 22.6372x; 1.0960x over previous
import functools
import math

import jax
import jax.numpy as jnp
from jax.experimental import pallas as pl
from jax.experimental.pallas import tpu as pltpu

NEG = -1e30


def _dot(a, b):
    return jax.lax.dot_general(a, b, (((1,), (0,)), ((), ())),
                               preferred_element_type=jnp.float32)


def _level_sizes(T):
    sizes = [T]
    n = T
    while n > 1:
        n = (n + 1) // 2
        sizes.append(n)
    return sizes


def _prep_kernel(mu_r, lam_r, pi_r, wq_r, bq_r, wk_r, bk_r, wv_r, bv_r,
                 ws1_r, bs1_r, ws2_r, bs2_r, wsr_r, bsr_r,
                 scr_r, scrT_r, q_o, k_o, v_o, s_o, sc_o, *, M, D):
    mu = mu_r[...]
    q = _dot(mu, wq_r[...]) + bq_r[...]
    k_o[...] = _dot(mu, wk_r[...]) + bk_r[...]
    v_o[...] = _dot(mu, wv_r[...]) + bv_r[...]
    q_o[...] = q

    lv = -jnp.log(jnp.maximum(lam_r[...], 1e-6))
    P7 = pi_r.shape[-1]
    h = (_dot(mu, ws1_r[0:D, :]) + _dot(lv, ws1_r[D:2 * D, :])
         + _dot(pi_r[...], ws1_r[2 * D:2 * D + P7, :]) + bs1_r[...])
    h = h * jax.nn.sigmoid(h)
    s = jax.nn.sigmoid(_dot(h, ws2_r[...]) + bs2_r[...])
    s_o[...] = s

    sq = _dot(q, wsr_r[...]) + bsr_r[...]
    sc_scores = _dot(sq, scrT_r[...]) * (1.0 / math.sqrt(M))
    m = jnp.max(sc_scores, axis=-1, keepdims=True)
    e = jnp.exp(sc_scores - m)
    attn = e / jnp.sum(e, axis=-1, keepdims=True)
    sc_o[...] = _dot(attn, scr_r[...])


def _tree_kernel(k_r, v_r, w_r, tkT_o, tv_o, tkA, tkB, tvA, tvB, w_s,
                 *, T, M, NTOT):
    MA = 128
    MB = M - MA
    k0 = k_r[0]
    v0 = v_r[0]
    tkA[0:T, :] = k0[:, 0:MA]
    tkB[0:T, 0:MB] = k0[:, MA:M]
    tvA[0:T, :] = v0[:, 0:MA]
    tvB[0:T, 0:MB] = v0[:, MA:M]
    w_s[0:T, :] = jnp.broadcast_to(w_r[0], (T, MA))
    tv_o[0, 0:T, :] = v0
    tkT_o[0, :, 0:T] = k0.T
    off = T
    prev = 0
    n = T
    while n > 1:
        h = n // 2
        w1 = w_s[prev:prev + n:2, :]
        w2 = w_s[prev + 1:prev + n:2, :]
        tot = w1 + w2 + 1e-8
        inv = 1.0 / tot
        nkA = (w1 * tkA[prev:prev + n:2, :]
               + w2 * tkA[prev + 1:prev + n:2, :]) * inv
        nkB = (w1 * tkB[prev:prev + n:2, :]
               + w2 * tkB[prev + 1:prev + n:2, :]) * inv
        nvA = (w1 * tvA[prev:prev + n:2, :]
               + w2 * tvA[prev + 1:prev + n:2, :]) * inv
        nvB = (w1 * tvB[prev:prev + n:2, :]
               + w2 * tvB[prev + 1:prev + n:2, :]) * inv
        tkA[off:off + h, :] = nkA
        tkB[off:off + h, :] = nkB
        tvA[off:off + h, :] = nvA
        tvB[off:off + h, :] = nvB
        w_s[off:off + h, :] = tot
        tv_o[0, off:off + h, 0:MA] = nvA
        tv_o[0, off:off + h, MA:M] = nvB[:, 0:MB]
        tkT_o[0, 0:MA, off:off + h] = nkA.T
        tkT_o[0, MA:M, off:off + h] = nkB[:, 0:MB].T
        prev = off
        off += h
        n = h
    if off < NTOT:
        tv_o[0, off:NTOT, :] = jnp.zeros((NTOT - off, M), jnp.float32)
        tkT_o[0, :, off:NTOT] = jnp.zeros((M, NTOT - off), jnp.float32)


def _attn_kernel(q_r, s_r, sc_r, tkT_r, tv_r, wo_r, bo_r, g_r, o_r, *,
                 TQ, M, Ns):
    i = pl.program_id(1)
    q = q_r[0]
    s = s_r[0]
    t_ids = jax.lax.broadcasted_iota(jnp.int32, (TQ, 1), 0) + i * TQ

    L = len(Ns)
    ss = s * float(L - 1)
    kaps = [jnp.exp(-0.5 * (float(l) - ss) ** 2) for l in range(L)]
    kden = kaps[0]
    for l in range(1, L):
        kden = kden + kaps[l]
    kinv = 1.0 / jnp.maximum(kden, 1e-8)

    q = q * (1.0 / math.sqrt(M))
    niota_full = jax.lax.broadcasted_iota(jnp.int32, (TQ, Ns[0]), 1)
    acc = jnp.zeros((TQ, M), jnp.float32)
    off = 0
    for l, N in enumerate(Ns):
        KT = tkT_r[0, :, off:off + N]
        V = tv_r[0, off:off + N, :]
        sc = jnp.where(niota_full[:, :N] > t_ids, NEG, _dot(q, KT))
        if N > 4:
            m1 = jnp.max(sc, axis=-1, keepdims=True)
            m2 = jnp.max(jnp.where(sc >= m1, NEG, sc), axis=-1, keepdims=True)
            m3 = jnp.max(jnp.where(sc >= m2, NEG, sc), axis=-1, keepdims=True)
            m4 = jnp.max(jnp.where(sc >= m3, NEG, sc), axis=-1, keepdims=True)
            e = jnp.exp(sc - m1)
            P = jnp.where(sc >= m4, e, 0.0)
            den = jnp.sum(P, axis=-1, keepdims=True)
            ctx = _dot(P, V)
        else:
            m = jnp.max(sc, axis=-1, keepdims=True)
            e = jnp.exp(sc - m)
            den = jnp.sum(e, axis=-1, keepdims=True)
            ctx = _dot(e, V)
        acc = acc + (kaps[l] * kinv / den) * ctx
        off += N

    g = jax.nn.sigmoid(g_r[0, 0])
    comb = (g * (1.0 - s)) * acc + (g * s) * sc_r[0]
    o_r[0] = _dot(comb, wo_r[...]) + g * bo_r[...]


def kernel(mu, lam, pi, pi_write, Wq, bq, Wk, bk, Wv, bv, Wo, bo,
           Ws1, bs1, Ws2, bs2, Wsr, bsr, scratch_init, gate):
    B, T, D = mu.shape
    M = Wq.shape[1]
    H = Ws1.shape[1]
    P7 = pi.shape[-1]
    BT = B * T
    Ns = _level_sizes(T)
    NTOT_raw = sum(Ns)
    NTOT = ((NTOT_raw + 7) // 8) * 8

    TQ = min(256, T)
    f32 = jnp.float32

    mu2 = mu.reshape(BT, D)
    lam2 = lam.reshape(BT, D)
    pi2 = pi.reshape(BT, P7)
    scr0 = scratch_init[0]
    scrT = scr0.T
    gate11 = gate.reshape(1, 1)

    def row_spec(w):
        return pl.BlockSpec((TQ, w), lambda i: (i, 0))

    def full_spec(shape):
        nd = len(shape)
        return pl.BlockSpec(shape, lambda i, _n=nd: (0,) * _n)

    grid_a = BT // TQ
    q2, k2, v2, s2, sc2 = pl.pallas_call(
        functools.partial(_prep_kernel, M=M, D=D),
        grid=(grid_a,),
        in_specs=[
            row_spec(D), row_spec(D), row_spec(P7),
            full_spec((D, M)), full_spec((1, M)),
            full_spec((D, M)), full_spec((1, M)),
            full_spec((D, M)), full_spec((1, M)),
            full_spec(Ws1.shape),
            full_spec((1, H)), full_spec((H, 1)), full_spec((1, 1)),
            full_spec((M, M)), full_spec((1, M)),
            full_spec(scr0.shape), full_spec(scrT.shape),
        ],
        out_specs=[row_spec(M), row_spec(M), row_spec(M), row_spec(1),
                   row_spec(M)],
        out_shape=[
            jax.ShapeDtypeStruct((BT, M), f32),
            jax.ShapeDtypeStruct((BT, M), f32),
            jax.ShapeDtypeStruct((BT, M), f32),
            jax.ShapeDtypeStruct((BT, 1), f32),
            jax.ShapeDtypeStruct((BT, M), f32),
        ],
    )(mu2, lam2, pi2,
      Wq, bq.reshape(1, M), Wk, bk.reshape(1, M), Wv, bv.reshape(1, M),
      Ws1, bs1.reshape(1, H), Ws2, bs2.reshape(1, 1),
      Wsr, bsr.reshape(1, M), scr0, scrT)

    k3 = k2.reshape(B, T, M)
    v3 = v2.reshape(B, T, M)
    w3 = pi_write

    tkT, tv = pl.pallas_call(
        functools.partial(_tree_kernel, T=T, M=M, NTOT=NTOT),
        grid=(B,),
        in_specs=[
            pl.BlockSpec((1, T, M), lambda b: (b, 0, 0)),
            pl.BlockSpec((1, T, M), lambda b: (b, 0, 0)),
            pl.BlockSpec((1, T, 1), lambda b: (b, 0, 0)),
        ],
        out_specs=[
            pl.BlockSpec((1, M, NTOT), lambda b: (b, 0, 0)),
            pl.BlockSpec((1, NTOT, M), lambda b: (b, 0, 0)),
        ],
        out_shape=[
            jax.ShapeDtypeStruct((B, M, NTOT), f32),
            jax.ShapeDtypeStruct((B, NTOT, M), f32),
        ],
        scratch_shapes=[
            pltpu.VMEM((NTOT, 128), f32),
            pltpu.VMEM((NTOT, 128), f32),
            pltpu.VMEM((NTOT, 128), f32),
            pltpu.VMEM((NTOT, 128), f32),
            pltpu.VMEM((NTOT, 128), f32),
        ],
    )(k3, v3, w3)

    q3 = q2.reshape(B, T, M)
    s3 = s2.reshape(B, T, 1)
    sc3 = sc2.reshape(B, T, M)

    out3 = pl.pallas_call(
        functools.partial(_attn_kernel, TQ=TQ, M=M, Ns=Ns),
        grid=(B, T // TQ),
        in_specs=[
            pl.BlockSpec((1, TQ, M), lambda b, i: (b, i, 0)),
            pl.BlockSpec((1, TQ, 1), lambda b, i: (b, i, 0)),
            pl.BlockSpec((1, TQ, M), lambda b, i: (b, i, 0)),
            pl.BlockSpec((1, M, NTOT), lambda b, i: (b, 0, 0)),
            pl.BlockSpec((1, NTOT, M), lambda b, i: (b, 0, 0)),
            pl.BlockSpec((M, D), lambda b, i: (0, 0)),
            pl.BlockSpec((1, D), lambda b, i: (0, 0)),
            pl.BlockSpec((1, 1), lambda b, i: (0, 0)),
        ],
        out_specs=pl.BlockSpec((1, TQ, D), lambda b, i: (b, i, 0)),
        out_shape=jax.ShapeDtypeStruct((B, T, D), f32),
    )(q3, s3, sc3, tkT, tv, Wo, bo.reshape(1, D), gate11)

    return out3, s2.reshape(B, T)
